# baseline (speedup 1.0000x reference)
import math

import jax
import jax.numpy as jnp
import numpy as np
from jax import lax
from jax.experimental import pallas as pl
from jax.experimental.pallas import tpu as pltpu
from jax.experimental.pallas import tpu_sc as plsc

EMBED_DIM = 1024
MAX_LENGTH = 2048
B, L = 4, 2048

TL = 2048


def _make_pos_enc_np():
    position = np.arange(MAX_LENGTH)[:, None].astype(np.float32)
    div_term = np.exp(
        np.arange(0, EMBED_DIM, 2).astype(np.float32)
        * (-math.log(10000.0) / EMBED_DIM)
    )
    pe = np.zeros((MAX_LENGTH, EMBED_DIM), dtype=np.float32)
    pe[:, 0::2] = np.sin(position * div_term)
    pe[:, 1::2] = np.cos(position * div_term)
    return pe


_POS_ENC = _make_pos_enc_np()[:L]


def _sc_gather_body(lvl_hbm, table_hbm, row_hbm, lvl_v, row_v, row_sem):
    cid = lax.axis_index("c")
    sid = lax.axis_index("s")

    @pl.when(jnp.logical_and(cid == 0, sid == 0))
    def _():
        pltpu.sync_copy(lvl_hbm, lvl_v)
        pltpu.async_copy(table_hbm.at[lvl_v], row_v, row_sem).wait()
        pltpu.sync_copy(row_v, row_hbm)


def _sc_gather(level, summary_table):
    lvl_arr = jnp.reshape(jnp.asarray(level, jnp.int32), (1,))
    mesh = plsc.VectorSubcoreMesh(core_axis_name="c", subcore_axis_name="s")
    fn = pl.kernel(
        _sc_gather_body,
        out_type=jax.ShapeDtypeStruct((1, EMBED_DIM), jnp.float32),
        mesh=mesh,
        scratch_types=[
            pltpu.VMEM((1,), jnp.int32),
            pltpu.VMEM((1, EMBED_DIM), jnp.float32),
            pltpu.SemaphoreType.DMA,
        ],
    )
    return fn(lvl_arr, summary_table)


def _tc_body(x_ref, pos_ref, row_ref, o_ref):
    o_ref[...] = x_ref[...] + pos_ref[...][None] + row_ref[...][None]


def kernel(x, level, summary_table):
    row = _sc_gather(level, summary_table)
    pos_enc = jnp.asarray(_POS_ENC)

    return pl.pallas_call(
        _tc_body,
        grid=(L // TL, B),
        in_specs=[
            pl.BlockSpec((1, TL, EMBED_DIM), lambda i, j: (j, i, 0)),
            pl.BlockSpec((TL, EMBED_DIM), lambda i, j: (i, 0)),
            pl.BlockSpec((1, EMBED_DIM), lambda i, j: (0, 0)),
        ],
        out_specs=pl.BlockSpec((1, TL, EMBED_DIM), lambda i, j: (j, i, 0)),
        out_shape=jax.ShapeDtypeStruct((B, L, EMBED_DIM), jnp.float32),
    )(x, pos_enc, row)


# speedup vs baseline: 1.0061x; 1.0061x over previous
import math

import jax
import jax.numpy as jnp
import numpy as np
from jax import lax
from jax.experimental import pallas as pl
from jax.experimental.pallas import tpu as pltpu
from jax.experimental.pallas import tpu_sc as plsc

EMBED_DIM = 1024
MAX_LENGTH = 2048
B, L = 4, 2048

TL = 2048


def _make_pos_enc_np():
    position = np.arange(MAX_LENGTH)[:, None].astype(np.float32)
    div_term = np.exp(
        np.arange(0, EMBED_DIM, 2).astype(np.float32)
        * (-math.log(10000.0) / EMBED_DIM)
    )
    pe = np.zeros((MAX_LENGTH, EMBED_DIM), dtype=np.float32)
    pe[:, 0::2] = np.sin(position * div_term)
    pe[:, 1::2] = np.cos(position * div_term)
    return pe


_POS_ENC = _make_pos_enc_np()[:L]


def _sc_gather_body(lvl_hbm, table_hbm, row_hbm, lvl_v, row_v, row_sem):
    cid = lax.axis_index("c")
    sid = lax.axis_index("s")

    @pl.when(jnp.logical_and(cid == 0, sid == 0))
    def _():
        pltpu.sync_copy(lvl_hbm, lvl_v)
        pltpu.async_copy(table_hbm.at[lvl_v], row_v, row_sem).wait()
        pltpu.sync_copy(row_v, row_hbm)


def _sc_gather(level, summary_table):
    lvl_arr = jnp.reshape(jnp.asarray(level, jnp.int32), (1,))
    mesh = plsc.VectorSubcoreMesh(core_axis_name="c", subcore_axis_name="s")
    fn = pl.kernel(
        _sc_gather_body,
        out_type=jax.ShapeDtypeStruct((1, EMBED_DIM), jnp.float32),
        mesh=mesh,
        scratch_types=[
            pltpu.VMEM((1,), jnp.int32),
            pltpu.VMEM((1, EMBED_DIM), jnp.float32),
            pltpu.SemaphoreType.DMA,
        ],
    )
    return fn(lvl_arr, summary_table)


def _tc_body(x_ref, pos_ref, row_ref, o_ref):
    o_ref[...] = x_ref[...] + pos_ref[...][None] + row_ref[...][None]


def _tc_head_body(lvl_ref, x_ref, pos_ref, table_ref, o_ref):
    row = table_ref[pl.ds(lvl_ref[0], 1), :]
    o_ref[...] = x_ref[...] + pos_ref[...][None] + row[None]


def _tc_tail_body(prev_ref, x_ref, pos_ref, row_ref, o_ref):
    del prev_ref
    o_ref[...] = x_ref[...] + pos_ref[...][None] + row_ref[...][None]


def kernel(x, level, summary_table):
    row = _sc_gather(level, summary_table)
    pos_enc = jnp.asarray(_POS_ENC)
    lvl_arr = jnp.reshape(jnp.asarray(level, jnp.int32), (1,))

    head = pl.pallas_call(
        _tc_head_body,
        grid=(B - 1,),
        in_specs=[
            pl.BlockSpec(memory_space=pltpu.SMEM),
            pl.BlockSpec((1, L, EMBED_DIM), lambda j: (j, 0, 0)),
            pl.BlockSpec((L, EMBED_DIM), lambda j: (0, 0)),
            pl.BlockSpec((16, EMBED_DIM), lambda j: (0, 0)),
        ],
        out_specs=pl.BlockSpec((1, L, EMBED_DIM), lambda j: (j, 0, 0)),
        out_shape=jax.ShapeDtypeStruct((B, L, EMBED_DIM), jnp.float32),
    )(lvl_arr, x, pos_enc, summary_table)

    return pl.pallas_call(
        _tc_tail_body,
        grid=(1,),
        in_specs=[
            pl.BlockSpec(memory_space=pl.ANY),
            pl.BlockSpec((1, L, EMBED_DIM), lambda j: (B - 1, 0, 0)),
            pl.BlockSpec((L, EMBED_DIM), lambda j: (0, 0)),
            pl.BlockSpec((1, EMBED_DIM), lambda j: (0, 0)),
        ],
        out_specs=pl.BlockSpec((1, L, EMBED_DIM), lambda j: (B - 1, 0, 0)),
        out_shape=jax.ShapeDtypeStruct((B, L, EMBED_DIM), jnp.float32),
        input_output_aliases={0: 0},
    )(head, x, pos_enc, row)


# speedup vs baseline: 1.0238x; 1.0176x over previous
import math

import jax
import jax.numpy as jnp
import numpy as np
from jax import lax
from jax.experimental import pallas as pl
from jax.experimental.pallas import tpu as pltpu
from jax.experimental.pallas import tpu_sc as plsc

EMBED_DIM = 1024
MAX_LENGTH = 2048
B, L = 4, 2048

TT = 256


def _make_pos_enc_np():
    position = np.arange(MAX_LENGTH)[:, None].astype(np.float32)
    div_term = np.exp(
        np.arange(0, EMBED_DIM, 2).astype(np.float32)
        * (-math.log(10000.0) / EMBED_DIM)
    )
    pe = np.zeros((MAX_LENGTH, EMBED_DIM), dtype=np.float32)
    pe[:, 0::2] = np.sin(position * div_term)
    pe[:, 1::2] = np.cos(position * div_term)
    return pe


_POS_ENC = _make_pos_enc_np()[:L]


def _sc_gather_body(lvl_hbm, table_hbm, row_hbm, lvl_v, row_v, row_sem):
    cid = lax.axis_index("c")
    sid = lax.axis_index("s")

    @pl.when(jnp.logical_and(cid == 0, sid == 0))
    def _():
        pltpu.sync_copy(lvl_hbm, lvl_v)
        pltpu.async_copy(table_hbm.at[lvl_v], row_v, row_sem).wait()
        pltpu.sync_copy(row_v, row_hbm)


def _sc_gather(level, summary_table):
    lvl_arr = jnp.reshape(jnp.asarray(level, jnp.int32), (1,))
    mesh = plsc.VectorSubcoreMesh(core_axis_name="c", subcore_axis_name="s")
    fn = pl.kernel(
        _sc_gather_body,
        out_type=jax.ShapeDtypeStruct((1, EMBED_DIM), jnp.float32),
        mesh=mesh,
        scratch_types=[
            pltpu.VMEM((1,), jnp.int32),
            pltpu.VMEM((1, EMBED_DIM), jnp.float32),
            pltpu.SemaphoreType.DMA,
        ],
    )
    return fn(lvl_arr, summary_table)


def _tc_body(x_ref, pos_ref, row_ref, o_ref):
    o_ref[...] = x_ref[...] + pos_ref[...][None] + row_ref[...][None]


def _tc_head_body(lvl_ref, x_ref, pos_ref, table_ref, o_ref):
    row = table_ref[pl.ds(lvl_ref[0], 1), :]
    o_ref[...] = x_ref[...] + pos_ref[...][None] + row[None]


def _tc_tail_body(prev_ref, x_ref, pos_ref, row_ref, o_ref):
    del prev_ref
    o_ref[...] = x_ref[...] + pos_ref[...][None] + row_ref[...][None]


def kernel(x, level, summary_table):
    row = _sc_gather(level, summary_table)
    pos_enc = jnp.asarray(_POS_ENC)
    lvl_arr = jnp.reshape(jnp.asarray(level, jnp.int32), (1,))

    LH = L - TT

    head = pl.pallas_call(
        _tc_head_body,
        grid=(B,),
        in_specs=[
            pl.BlockSpec(memory_space=pltpu.SMEM),
            pl.BlockSpec((1, LH, EMBED_DIM), lambda j: (j, 0, 0)),
            pl.BlockSpec((LH, EMBED_DIM), lambda j: (0, 0)),
            pl.BlockSpec((16, EMBED_DIM), lambda j: (0, 0)),
        ],
        out_specs=pl.BlockSpec((1, LH, EMBED_DIM), lambda j: (j, 0, 0)),
        out_shape=jax.ShapeDtypeStruct((B, L, EMBED_DIM), jnp.float32),
    )(lvl_arr, x, pos_enc, summary_table)

    return pl.pallas_call(
        _tc_tail_body,
        grid=(B,),
        in_specs=[
            pl.BlockSpec(memory_space=pl.ANY),
            pl.BlockSpec((1, TT, EMBED_DIM), lambda j: (j, LH // TT, 0)),
            pl.BlockSpec((TT, EMBED_DIM), lambda j: (LH // TT, 0)),
            pl.BlockSpec((1, EMBED_DIM), lambda j: (0, 0)),
        ],
        out_specs=pl.BlockSpec((1, TT, EMBED_DIM), lambda j: (j, LH // TT, 0)),
        out_shape=jax.ShapeDtypeStruct((B, L, EMBED_DIM), jnp.float32),
        input_output_aliases={0: 0},
    )(head, x, pos_enc, row)


# speedup vs baseline: 1.0289x; 1.0050x over previous
import math

import jax
import jax.numpy as jnp
import numpy as np
from jax import lax
from jax.experimental import pallas as pl
from jax.experimental.pallas import tpu as pltpu
from jax.experimental.pallas import tpu_sc as plsc

EMBED_DIM = 1024
MAX_LENGTH = 2048
B, L = 4, 2048

TT = 256


def _make_pos_enc_np():
    position = np.arange(MAX_LENGTH)[:, None].astype(np.float32)
    div_term = np.exp(
        np.arange(0, EMBED_DIM, 2).astype(np.float32)
        * (-math.log(10000.0) / EMBED_DIM)
    )
    pe = np.zeros((MAX_LENGTH, EMBED_DIM), dtype=np.float32)
    pe[:, 0::2] = np.sin(position * div_term)
    pe[:, 1::2] = np.cos(position * div_term)
    return pe


_POS_ENC = _make_pos_enc_np()[:L]


def _sc_gather_body(lvl_hbm, table_hbm, row_hbm, lvl_v, row_v, row_sem):
    cid = lax.axis_index("c")
    sid = lax.axis_index("s")

    @pl.when(jnp.logical_and(cid == 0, sid == 0))
    def _():
        pltpu.sync_copy(lvl_hbm, lvl_v)
        pltpu.async_copy(table_hbm.at[lvl_v], row_v, row_sem).wait()
        pltpu.sync_copy(row_v, row_hbm)


def _sc_gather(level, summary_table):
    lvl_arr = jnp.reshape(jnp.asarray(level, jnp.int32), (1,))
    mesh = plsc.VectorSubcoreMesh(core_axis_name="c", subcore_axis_name="s")
    fn = pl.kernel(
        _sc_gather_body,
        out_type=jax.ShapeDtypeStruct((1, EMBED_DIM), jnp.float32),
        mesh=mesh,
        scratch_types=[
            pltpu.VMEM((1,), jnp.int32),
            pltpu.VMEM((1, EMBED_DIM), jnp.float32),
            pltpu.SemaphoreType.DMA,
        ],
    )
    return fn(lvl_arr, summary_table)


def _tc_head_body(lvl_ref, x_ref, pos_ref, table_ref, o_ref):
    row = table_ref[pl.ds(lvl_ref[0], 1), :]
    o_ref[...] = x_ref[...] + pos_ref[...][None] + row[None]


def _tc_tail_body(prev_ref, x_ref, pos_ref, row_ref, o_ref):
    del prev_ref
    o_ref[...] = x_ref[...] + pos_ref[...][None] + row_ref[...][None]


def kernel(x, level, summary_table):
    row = _sc_gather(level, summary_table)
    pos_enc = jnp.asarray(_POS_ENC)
    lvl_arr = jnp.reshape(jnp.asarray(level, jnp.int32), (1,))

    LH = L - TT

    head = pl.pallas_call(
        _tc_head_body,
        grid=(B,),
        in_specs=[
            pl.BlockSpec(memory_space=pltpu.SMEM),
            pl.BlockSpec((1, LH, EMBED_DIM), lambda j: (j, 0, 0)),
            pl.BlockSpec((LH, EMBED_DIM), lambda j: (0, 0)),
            pl.BlockSpec((16, EMBED_DIM), lambda j: (0, 0)),
        ],
        out_specs=pl.BlockSpec((1, LH, EMBED_DIM), lambda j: (j, 0, 0)),
        out_shape=jax.ShapeDtypeStruct((B, L, EMBED_DIM), jnp.float32),
    )(lvl_arr, x, pos_enc, summary_table)

    return pl.pallas_call(
        _tc_tail_body,
        grid=(B,),
        in_specs=[
            pl.BlockSpec(memory_space=pl.ANY),
            pl.BlockSpec((1, TT, EMBED_DIM), lambda j: (j, LH // TT, 0)),
            pl.BlockSpec((TT, EMBED_DIM), lambda j: (LH // TT, 0)),
            pl.BlockSpec((1, EMBED_DIM), lambda j: (0, 0)),
        ],
        out_specs=pl.BlockSpec((1, TT, EMBED_DIM), lambda j: (j, LH // TT, 0)),
        out_shape=jax.ShapeDtypeStruct((B, L, EMBED_DIM), jnp.float32),
        input_output_aliases={0: 0},
    )(head, x, pos_enc, row)
